# baseline (speedup 1.0000x reference)
import functools

import jax
import jax.numpy as jnp
from jax import lax
from jax.experimental import pallas as pl
from jax.experimental.pallas import tpu as pltpu
from jax.experimental.pallas import tpu_sc as plsc

_NUM_WORKERS = 32
_ROWS = 800
_NBUF = 2


def _pos_table(seq_len, model_dim):
    pos = jnp.arange(seq_len, dtype=jnp.float32)[:, None]
    two_i = jnp.arange(0, model_dim, 2, dtype=jnp.float32)
    angles = pos / (10000.0 ** (two_i / model_dim))
    enc = jnp.zeros((seq_len, model_dim), dtype=jnp.float32)
    enc = enc.at[:, 0::2].set(jnp.sin(angles))
    enc = enc.at[:, 1::2].set(jnp.cos(angles))
    return enc


def _make_sc_kernel(n_tokens, seq_len, model_dim):
    tok_per_w = n_tokens // _NUM_WORKERS
    cpw = tok_per_w // _ROWS
    reps = _ROWS // seq_len
    mesh = plsc.VectorSubcoreMesh(core_axis_name="c", subcore_axis_name="s")

    @functools.partial(
        pl.kernel,
        out_type=jax.ShapeDtypeStruct((n_tokens, model_dim), jnp.float32),
        mesh=mesh,
        scratch_types=[
            pltpu.VMEM((tok_per_w,), jnp.int32),
            pltpu.VMEM((seq_len, model_dim), jnp.float32),
            [pltpu.VMEM((_ROWS, model_dim), jnp.float32)
             for _ in range(_NBUF)],
            [pltpu.SemaphoreType.DMA for _ in range(_NBUF)],
            [pltpu.SemaphoreType.DMA for _ in range(_NBUF)],
        ],
        compiler_params=pltpu.CompilerParams(use_tc_tiling_on_sc=False),
    )
    def emb_kernel(idx_hbm, table_hbm, pos_hbm, out_hbm, idx_all, pos_v, rows,
                   gsems, ssems):
        wid = lax.axis_index("s") * 2 + lax.axis_index("c")
        pltpu.sync_copy(idx_hbm.at[pl.ds(wid * tok_per_w, tok_per_w)],
                        idx_all)
        pltpu.sync_copy(pos_hbm, pos_v)

        def issue_gather(j, b):
            pltpu.async_copy(
                table_hbm.at[idx_all.at[pl.ds(j * _ROWS, _ROWS)]],
                rows[b], gsems[b])

        def wait_gather(b):
            pltpu.make_async_copy(
                table_hbm.at[idx_all.at[pl.ds(0, _ROWS)]],
                rows[b], gsems[b]).wait()

        def issue_store(j, b):
            pltpu.async_copy(
                rows[b],
                out_hbm.at[pl.ds((wid * cpw + j) * _ROWS, _ROWS)],
                ssems[b])

        def wait_store(b):
            pltpu.make_async_copy(
                rows[b], out_hbm.at[pl.ds(0, _ROWS)], ssems[b]).wait()

        def add_pos(b):
            rv = rows[b]
            for r in range(reps):

                def tok(i, carry, _base=r * seq_len):
                    for c in range(model_dim // 16):
                        plsc.addupdate(
                            rv.at[_base + i, pl.ds(16 * c, 16)],
                            pos_v[i, pl.ds(16 * c, 16)])
                    return carry

                lax.fori_loop(0, seq_len, tok, 0, unroll=4)

        issue_gather(0, 0)
        for j in range(cpw):
            b = j % _NBUF
            nb = (j + 1) % _NBUF
            if j + 1 < cpw:
                if j >= _NBUF - 1:
                    wait_store(nb)
                issue_gather(j + 1, nb)
            wait_gather(b)
            add_pos(b)
            issue_store(j, b)
        for k in range(_NBUF):
            wait_store((cpw - _NBUF + 1 + k) % _NBUF)

    return emb_kernel


@jax.jit
def kernel(x, table):
    batch, seq_len = x.shape
    model_dim = table.shape[1]
    n_tokens = batch * seq_len
    idx_flat = x.reshape(n_tokens).astype(jnp.int32)
    pos = _pos_table(seq_len, model_dim)
    out_flat = _make_sc_kernel(n_tokens, seq_len, model_dim)(
        idx_flat, table, pos)
    return out_flat.reshape(batch, seq_len, model_dim)


# speedup vs baseline: 1.0073x; 1.0073x over previous
import functools

import jax
import jax.numpy as jnp
from jax import lax
from jax.experimental import pallas as pl
from jax.experimental.pallas import tpu as pltpu
from jax.experimental.pallas import tpu_sc as plsc

_NUM_WORKERS = 32
_ROWS = 200
_NBUF = 4


def _pos_table(seq_len, model_dim):
    pos = jnp.arange(seq_len, dtype=jnp.float32)[:, None]
    two_i = jnp.arange(0, model_dim, 2, dtype=jnp.float32)
    angles = pos / (10000.0 ** (two_i / model_dim))
    enc = jnp.zeros((seq_len, model_dim), dtype=jnp.float32)
    enc = enc.at[:, 0::2].set(jnp.sin(angles))
    enc = enc.at[:, 1::2].set(jnp.cos(angles))
    return enc


def _make_sc_kernel(n_tokens, seq_len, model_dim):
    tok_per_w = n_tokens // _NUM_WORKERS
    cpw = tok_per_w // _ROWS
    reps = _ROWS // seq_len
    mesh = plsc.VectorSubcoreMesh(core_axis_name="c", subcore_axis_name="s")

    @functools.partial(
        pl.kernel,
        out_type=jax.ShapeDtypeStruct((n_tokens, model_dim), jnp.float32),
        mesh=mesh,
        scratch_types=[
            pltpu.VMEM((tok_per_w,), jnp.int32),
            pltpu.VMEM((seq_len, model_dim), jnp.float32),
            [pltpu.VMEM((_ROWS, model_dim), jnp.float32)
             for _ in range(_NBUF)],
            [pltpu.SemaphoreType.DMA for _ in range(_NBUF)],
            [pltpu.SemaphoreType.DMA for _ in range(_NBUF)],
        ],
        compiler_params=pltpu.CompilerParams(use_tc_tiling_on_sc=False),
    )
    def emb_kernel(idx_hbm, table_hbm, pos_hbm, out_hbm, idx_all, pos_v, rows,
                   gsems, ssems):
        wid = lax.axis_index("s") * 2 + lax.axis_index("c")
        pltpu.sync_copy(idx_hbm.at[pl.ds(wid * tok_per_w, tok_per_w)],
                        idx_all)
        pltpu.sync_copy(pos_hbm, pos_v)

        def issue_gather(j, b):
            pltpu.async_copy(
                table_hbm.at[idx_all.at[pl.ds(j * _ROWS, _ROWS)]],
                rows[b], gsems[b])

        def wait_gather(b):
            pltpu.make_async_copy(
                table_hbm.at[idx_all.at[pl.ds(0, _ROWS)]],
                rows[b], gsems[b]).wait()

        def issue_store(j, b):
            pltpu.async_copy(
                rows[b],
                out_hbm.at[pl.ds((wid * cpw + j) * _ROWS, _ROWS)],
                ssems[b])

        def wait_store(b):
            pltpu.make_async_copy(
                rows[b], out_hbm.at[pl.ds(0, _ROWS)], ssems[b]).wait()

        def add_pos(b):
            rv = rows[b]
            for r in range(reps):

                def tok(i, carry, _base=r * seq_len):
                    for c in range(model_dim // 16):
                        plsc.addupdate(
                            rv.at[_base + i, pl.ds(16 * c, 16)],
                            pos_v[i, pl.ds(16 * c, 16)])
                    return carry

                lax.fori_loop(0, seq_len, tok, 0, unroll=4)

        issue_gather(0, 0)
        for j in range(cpw):
            b = j % _NBUF
            nb = (j + 1) % _NBUF
            if j + 1 < cpw:
                if j >= _NBUF - 1:
                    wait_store(nb)
                issue_gather(j + 1, nb)
            wait_gather(b)
            add_pos(b)
            issue_store(j, b)
        for k in range(_NBUF):
            wait_store((cpw - _NBUF + 1 + k) % _NBUF)

    return emb_kernel


@jax.jit
def kernel(x, table):
    batch, seq_len = x.shape
    model_dim = table.shape[1]
    n_tokens = batch * seq_len
    idx_flat = x.reshape(n_tokens).astype(jnp.int32)
    pos = _pos_table(seq_len, model_dim)
    out_flat = _make_sc_kernel(n_tokens, seq_len, model_dim)(
        idx_flat, table, pos)
    return out_flat.reshape(batch, seq_len, model_dim)
